# baseline (speedup 1.0000x reference)
import functools

import jax
import jax.numpy as jnp
from jax.experimental import pallas as pl
from jax.experimental.pallas import tpu as pltpu


def _pack_weights(D, W):
    d = jax.lax.broadcasted_iota(jnp.int32, (D, 2 * W), 0)
    col = jax.lax.broadcasted_iota(jnp.int32, (D, 2 * W), 1)
    word = d % W
    bit = d // W
    half = bit // 16
    hit = (col % W == word) & (col // W == half)
    val = (jnp.int32(1) << (bit % 16)).astype(jnp.float32)
    return jnp.where(hit, val, 0.0)


def _quant_body(x_ref, idx_ref, *, L):
    x = x_ref[...]
    idx_ref[...] = jnp.clip(jnp.round(x * (L - 1)), 0.0, L - 1.0).astype(jnp.int32)


def _csa_popcount_lt13(words):
    pools = {0: list(words)}
    planes = {}
    w = 0
    while pools.get(w):
        pool = pools[w]
        while len(pool) >= 3:
            a, b, cn = pool.pop(), pool.pop(), pool.pop()
            t = a ^ b
            s = t ^ cn
            carry = (a & b) | (cn & t)
            pool.append(s)
            pools.setdefault(w + 1, []).append(carry)
        if len(pool) == 2:
            a, b = pool.pop(), pool.pop()
            pool.append(a ^ b)
            pools.setdefault(w + 1, []).append(a & b)
        planes[w] = pool[0]
        w += 1
    z = jnp.zeros_like(words[0])
    p = [planes.get(i, z) for i in range(5)]
    return ~p[4] & (~p[3] | ~p[2] | (~p[1] & ~p[0]))


def _main_body(idx_ref, lw_ref, keys_ref, out_ref, pb_ref, kb_ref, asm_ref,
               *, C, W):
    @pl.when(pl.program_id(0) == 0)
    def _pack_tables():
        D = lw_ref.shape[1]
        m = _pack_weights(D, W).astype(jnp.bfloat16)
        lw_bits = (lw_ref[...] < 0).astype(jnp.bfloat16)
        pk = jnp.dot(lw_bits, m, preferred_element_type=jnp.float32)
        pb_ref[...] = (pk[:, :W].astype(jnp.int32)
                       | (pk[:, W:].astype(jnp.int32) << 16))
        k_bits = (keys_ref[...] < 0).astype(jnp.bfloat16)
        kk = jnp.dot(k_bits, m, preferred_element_type=jnp.float32)
        kb_ref[...] = (kk[:, :W].astype(jnp.int32)
                       | (kk[:, W:].astype(jnp.int32) << 16))

    BB = out_ref.shape[0]
    for g in range(BB // 8):
        words = []
        for c in range(C):
            slot = c % 26
            for s in range(8):
                r = idx_ref[g * 8 + s, c]
                asm_ref[slot, s, :] = pb_ref[r, :]
            kb_c = jnp.broadcast_to(kb_ref[c:c + 1, :], (8, W))
            words.append(asm_ref[slot] ^ kb_c)
        pos = _csa_popcount_lt13(words)
        for m in range(32):
            bit = (pos << (31 - m)) < 0
            out_ref[g * 8:(g + 1) * 8, m * W:(m + 1) * W] = (
                jnp.where(bit, 1.0, -1.0))


@jax.jit
def kernel(input, keys_weight, level_weight):
    B, C = input.shape
    L, D = level_weight.shape
    W = 128
    idx = pl.pallas_call(
        functools.partial(_quant_body, L=L),
        grid=(1,),
        in_specs=[pl.BlockSpec((B, C), lambda i: (0, 0))],
        out_specs=pl.BlockSpec((B, C), lambda i: (0, 0)),
        out_shape=jax.ShapeDtypeStruct((B, C), jnp.int32),
    )(input)

    BB = 128
    out = pl.pallas_call(
        functools.partial(_main_body, C=C, W=W),
        grid=(B // BB,),
        in_specs=[
            pl.BlockSpec((BB, C), lambda i: (i, 0), memory_space=pltpu.SMEM),
            pl.BlockSpec((L, D), lambda i: (0, 0)),
            pl.BlockSpec((C, D), lambda i: (0, 0)),
        ],
        out_specs=pl.BlockSpec((BB, D), lambda i: (i, 0)),
        out_shape=jax.ShapeDtypeStruct((B, D), jnp.float32),
        scratch_shapes=[
            pltpu.VMEM((L, W), jnp.int32),
            pltpu.VMEM((C, W), jnp.int32),
            pltpu.VMEM((26, 8, W), jnp.int32),
        ],
    )(idx, level_weight, keys_weight)
    return out


# speedup vs baseline: 1.0031x; 1.0031x over previous
import functools

import jax
import jax.numpy as jnp
from jax.experimental import pallas as pl
from jax.experimental.pallas import tpu as pltpu


def _pack_weights(D, W):
    d = jax.lax.broadcasted_iota(jnp.int32, (D, 2 * W), 0)
    col = jax.lax.broadcasted_iota(jnp.int32, (D, 2 * W), 1)
    word = d % W
    bit = d // W
    half = bit // 16
    hit = (col % W == word) & (col // W == half)
    val = (jnp.int32(1) << (bit % 16)).astype(jnp.float32)
    return jnp.where(hit, val, 0.0)


def _quant_body(x_ref, idx_ref, *, L):
    x = x_ref[...]
    idx_ref[...] = jnp.clip(jnp.round(x * (L - 1)), 0.0, L - 1.0).astype(jnp.int32)


def _csa_popcount_lt13(words):
    pools = {0: list(words)}
    planes = {}
    w = 0
    while pools.get(w):
        pool = pools[w]
        while len(pool) >= 3:
            a, b, cn = pool.pop(), pool.pop(), pool.pop()
            t = a ^ b
            s = t ^ cn
            carry = (a & b) | (cn & t)
            pool.append(s)
            pools.setdefault(w + 1, []).append(carry)
        if len(pool) == 2:
            a, b = pool.pop(), pool.pop()
            pool.append(a ^ b)
            pools.setdefault(w + 1, []).append(a & b)
        planes[w] = pool[0]
        w += 1
    z = jnp.zeros_like(words[0])
    p = [planes.get(i, z) for i in range(5)]
    return ~p[4] & (~p[3] | ~p[2] | (~p[1] & ~p[0]))


def _main_body(idx_ref, lw_ref, keys_ref, out_ref, pb_ref, kb_ref, asm_ref,
               *, C, W):
    @pl.when(pl.program_id(0) == 0)
    def _pack_tables():
        D = lw_ref.shape[1]
        m = _pack_weights(D, W).astype(jnp.bfloat16)
        lw_bits = (lw_ref[...] < 0).astype(jnp.bfloat16)
        pk = jnp.dot(lw_bits, m, preferred_element_type=jnp.float32)
        pb_ref[...] = (pk[:, :W].astype(jnp.int32)
                       | (pk[:, W:].astype(jnp.int32) << 16))
        k_bits = (keys_ref[...] < 0).astype(jnp.bfloat16)
        kk = jnp.dot(k_bits, m, preferred_element_type=jnp.float32)
        kb_ref[...] = (kk[:, :W].astype(jnp.int32)
                       | (kk[:, W:].astype(jnp.int32) << 16))

    BB = out_ref.shape[0]
    for g in range(BB // 8):
        words = []
        for c in range(C):
            slot = (g * 26 + c) % 16
            for s in range(8):
                r = idx_ref[g * 8 + s, c]
                asm_ref[slot, s, :] = pb_ref[r, :]
            kb_c = jnp.broadcast_to(kb_ref[c:c + 1, :], (8, W))
            words.append(asm_ref[slot] ^ kb_c)
        pos = _csa_popcount_lt13(words)
        for m in range(32):
            bit = (pos << (31 - m)) < 0
            out_ref[g * 8:(g + 1) * 8, m * W:(m + 1) * W] = (
                jnp.where(bit, 1.0, -1.0))


@jax.jit
def kernel(input, keys_weight, level_weight):
    B, C = input.shape
    L, D = level_weight.shape
    W = 128
    idx = pl.pallas_call(
        functools.partial(_quant_body, L=L),
        grid=(1,),
        in_specs=[pl.BlockSpec((B, C), lambda i: (0, 0))],
        out_specs=pl.BlockSpec((B, C), lambda i: (0, 0)),
        out_shape=jax.ShapeDtypeStruct((B, C), jnp.int32),
    )(input)

    BB = 128
    out = pl.pallas_call(
        functools.partial(_main_body, C=C, W=W),
        grid=(B // BB,),
        in_specs=[
            pl.BlockSpec((BB, C), lambda i: (i, 0), memory_space=pltpu.SMEM),
            pl.BlockSpec((L, D), lambda i: (0, 0)),
            pl.BlockSpec((C, D), lambda i: (0, 0)),
        ],
        out_specs=pl.BlockSpec((BB, D), lambda i: (i, 0)),
        out_shape=jax.ShapeDtypeStruct((B, D), jnp.float32),
        scratch_shapes=[
            pltpu.VMEM((L, W), jnp.int32),
            pltpu.VMEM((C, W), jnp.int32),
            pltpu.VMEM((16, 8, W), jnp.int32),
        ],
    )(idx, level_weight, keys_weight)
    return out


# speedup vs baseline: 1.0626x; 1.0594x over previous
import functools

import jax
import jax.numpy as jnp
from jax.experimental import pallas as pl
from jax.experimental.pallas import tpu as pltpu


def _pack_weights(D, W):
    d = jax.lax.broadcasted_iota(jnp.int32, (D, 2 * W), 0)
    col = jax.lax.broadcasted_iota(jnp.int32, (D, 2 * W), 1)
    word = d % W
    bit = d // W
    half = bit // 16
    hit = (col % W == word) & (col // W == half)
    val = (jnp.int32(1) << (bit % 16)).astype(jnp.float32)
    return jnp.where(hit, val, 0.0)


def _quant_body(x_ref, idx_ref, *, L):
    x = x_ref[...]
    idx_ref[...] = jnp.clip(jnp.round(x * (L - 1)), 0.0, L - 1.0).astype(jnp.int32)


def _csa_popcount_lt13(words):
    pools = {0: list(words)}
    planes = {}
    w = 0
    while pools.get(w):
        pool = pools[w]
        while len(pool) >= 3:
            a, b, cn = pool.pop(), pool.pop(), pool.pop()
            t = a ^ b
            s = t ^ cn
            carry = (a & b) | (cn & t)
            pool.append(s)
            pools.setdefault(w + 1, []).append(carry)
        if len(pool) == 2:
            a, b = pool.pop(), pool.pop()
            pool.append(a ^ b)
            pools.setdefault(w + 1, []).append(a & b)
        planes[w] = pool[0]
        w += 1
    z = jnp.zeros_like(words[0])
    p = [planes.get(i, z) for i in range(5)]
    return ~p[4] & (~p[3] | ~p[2] | (~p[1] & ~p[0]))


def _main_body(idx_ref, lw_ref, keys_ref, out_ref, pb_ref, kb_ref, asm_ref,
               *, C, W):
    @pl.when(pl.program_id(0) == 0)
    def _pack_tables():
        D = lw_ref.shape[1]
        m = _pack_weights(D, W).astype(jnp.bfloat16)
        lw_bits = (lw_ref[...] < 0).astype(jnp.bfloat16)
        pk = jnp.dot(lw_bits, m, preferred_element_type=jnp.float32)
        pb_ref[...] = (pk[:, :W].astype(jnp.int32)
                       | (pk[:, W:].astype(jnp.int32) << 16))
        k_bits = (keys_ref[...] < 0).astype(jnp.bfloat16)
        kk = jnp.dot(k_bits, m, preferred_element_type=jnp.float32)
        kb_ref[...] = (kk[:, :W].astype(jnp.int32)
                       | (kk[:, W:].astype(jnp.int32) << 16))

    BB = out_ref.shape[0]
    for g in range(BB // 8):
        words = []
        for c in range(C):
            slot = c % 16
            for s in range(8):
                r = idx_ref[g * 8 + s, c]
                asm_ref[slot, s, :] = pb_ref[r, :]
            kb_c = jnp.broadcast_to(kb_ref[c:c + 1, :], (8, W))
            words.append(asm_ref[slot] ^ kb_c)
        pos = _csa_popcount_lt13(words)
        for m in range(32):
            bit = (pos << (31 - m)) < 0
            out_ref[g * 8:(g + 1) * 8, m * W:(m + 1) * W] = (
                jnp.where(bit, 1.0, -1.0))


@jax.jit
def kernel(input, keys_weight, level_weight):
    B, C = input.shape
    L, D = level_weight.shape
    W = 128
    idx = pl.pallas_call(
        functools.partial(_quant_body, L=L),
        grid=(1,),
        in_specs=[pl.BlockSpec((B, C), lambda i: (0, 0))],
        out_specs=pl.BlockSpec((B, C), lambda i: (0, 0)),
        out_shape=jax.ShapeDtypeStruct((B, C), jnp.int32),
    )(input)

    BB = 128
    out = pl.pallas_call(
        functools.partial(_main_body, C=C, W=W),
        grid=(B // BB,),
        in_specs=[
            pl.BlockSpec((BB, C), lambda i: (i, 0), memory_space=pltpu.SMEM),
            pl.BlockSpec((L, D), lambda i: (0, 0)),
            pl.BlockSpec((C, D), lambda i: (0, 0)),
        ],
        out_specs=pl.BlockSpec((BB, D), lambda i: (i, 0)),
        out_shape=jax.ShapeDtypeStruct((B, D), jnp.float32),
        scratch_shapes=[
            pltpu.VMEM((L, W), jnp.int32),
            pltpu.VMEM((C, W), jnp.int32),
            pltpu.VMEM((16, 8, W), jnp.int32),
        ],
    )(idx, level_weight, keys_weight)
    return out
